# baseline (speedup 1.0000x reference)
import jax
import jax.numpy as jnp
from jax import lax
from jax.experimental import pallas as pl
from jax.experimental.pallas import tpu as pltpu

_B = 128
_V = 100000
_K = 1024

_TILE = 16384
_NT = -(-_V // _TILE)


def _tc_stream_body(x_ref, ids_ref, bounds_ref, tok_ref, bg_ref,
                    m_out, s_out, topv_ref, xtok_out, m_s, s_s, tok_s, acc_s):
    i = pl.program_id(0)
    c = bg_ref[2]

    @pl.when(i == 0)
    def _init():
        m_s[...] = jnp.full((_B, 128), -jnp.inf, jnp.float32)
        s_s[...] = jnp.zeros((_B, 128), jnp.float32)
        tok_s[...] = jnp.zeros((_B, 128), jnp.float32)
        acc_s[...] = jnp.zeros((_B, 128), jnp.float32)

    @pl.when(i == _NT - 1)
    def _mask_tail():
        x_ref[:, _V % _TILE:] = jnp.full(
            (_B, _TILE - _V % _TILE), -3.0e38, jnp.float32)

    lo = bounds_ref[i]
    hi = bounds_ref[i + 1]

    lane = lax.broadcasted_iota(jnp.int32, (1, 128), 1)

    def _grab(k, acc):
        local = ids_ref[k] - i * _TILE
        base = pl.multiple_of((local // 128) * 128, 128)
        r = local - base
        chunk = x_ref[:, pl.ds(base, 128)]
        kl = k % 128
        rolled = pltpu.roll(chunk, kl - r, axis=1)
        acc = acc + jnp.where(lane == kl, rolled, 0.0)

        def _flush(a):
            topv_ref[:, pl.ds(pl.multiple_of(k - 127, 128), 128)] = a
            return jnp.zeros_like(a)

        return lax.cond((k + 1) % 128 == 0, _flush, lambda a: a, acc)

    acc_s[...] = lax.fori_loop(lo, hi, _grab, acc_s[...])

    a = x_ref[...] * c
    col = lax.broadcasted_iota(jnp.int32, (_B, _TILE), 1)

    local_tok = tok_ref[...] - i * _TILE
    contrib = jnp.sum(jnp.where(col == local_tok, a, 0.0),
                      axis=1, keepdims=True)
    tok_s[...] = tok_s[...] + contrib

    m_old = m_s[...][:, :1]
    s_old = s_s[...][:, :1]
    m_new = jnp.maximum(m_old, jnp.max(a, axis=1, keepdims=True))
    s_new = s_old * jnp.exp2(m_old - m_new) + jnp.sum(
        jnp.exp2(a - m_new), axis=1, keepdims=True)
    m_s[...] = jnp.broadcast_to(m_new, (_B, 128))
    s_s[...] = jnp.broadcast_to(s_new, (_B, 128))

    @pl.when(i == _NT - 1)
    def _emit():
        m_out[...] = m_new
        s_out[...] = s_new
        xtok_out[...] = tok_s[...][:, :1]


_tc_stream = pl.pallas_call(
    _tc_stream_body,
    grid=(_NT,),
    in_specs=[
        pl.BlockSpec((_B, _TILE), lambda i: (0, i)),
        pl.BlockSpec(memory_space=pltpu.SMEM),
        pl.BlockSpec(memory_space=pltpu.SMEM),
        pl.BlockSpec((_B, 1), lambda i: (0, 0)),
        pl.BlockSpec(memory_space=pltpu.SMEM),
    ],
    out_specs=[
        pl.BlockSpec((_B, 1), lambda i: (0, 0)),
        pl.BlockSpec((_B, 1), lambda i: (0, 0)),
        pl.BlockSpec((_B, _K), lambda i: (0, 0)),
        pl.BlockSpec((_B, 1), lambda i: (0, 0)),
    ],
    out_shape=[
        jax.ShapeDtypeStruct((_B, 1), jnp.float32),
        jax.ShapeDtypeStruct((_B, 1), jnp.float32),
        jax.ShapeDtypeStruct((_B, _K), jnp.float32),
        jax.ShapeDtypeStruct((_B, 1), jnp.float32),
    ],
    scratch_shapes=[
        pltpu.VMEM((_B, 128), jnp.float32),
        pltpu.VMEM((_B, 128), jnp.float32),
        pltpu.VMEM((_B, 128), jnp.float32),
        pltpu.VMEM((_B, 128), jnp.float32),
    ],
    compiler_params=pltpu.CompilerParams(
        dimension_semantics=("arbitrary",)),
)


def _tc_epi_body(top_ref, w_ref, ids_ref, tok_ref, xtok_ref, m_ref, s_ref,
                 bg_ref, out_ref):
    inv_g = bg_ref[1]
    c = bg_ref[2]
    a_top = top_ref[...] * inv_g
    a_bf = a_top.astype(jnp.bfloat16).astype(jnp.float32)
    w_bf = w_ref[...].astype(jnp.bfloat16).astype(jnp.float32)
    t = jnp.clip(
        jnp.sum(a_bf * w_bf, axis=1, keepdims=True) + bg_ref[0],
        1e-6, None)
    a2_top = top_ref[...] * c
    top_mod = a2_top / t
    m_all = m_ref[...]
    s_all = s_ref[...]
    m2 = jnp.maximum(m_all, jnp.max(top_mod, axis=1, keepdims=True))
    corr = jnp.sum(jnp.exp2(top_mod - m2) - jnp.exp2(a2_top - m2),
                   axis=1, keepdims=True)
    s_tot = s_all * jnp.exp2(m_all - m2) + corr
    in_top = jnp.sum(
        jnp.where(ids_ref[...] == tok_ref[...], 1.0, 0.0),
        axis=1, keepdims=True)
    a_tok = xtok_ref[...]
    a_eff = jnp.where(in_top > 0.0, a_tok / t, a_tok)
    out_ref[...] = jnp.exp2(a_eff - m2) / s_tot


_tc_epi = pl.pallas_call(
    _tc_epi_body,
    in_specs=[
        pl.BlockSpec((_B, _K), lambda: (0, 0)),
        pl.BlockSpec((1, _K), lambda: (0, 0)),
        pl.BlockSpec((1, _K), lambda: (0, 0)),
        pl.BlockSpec((_B, 1), lambda: (0, 0)),
        pl.BlockSpec((_B, 1), lambda: (0, 0)),
        pl.BlockSpec((_B, 1), lambda: (0, 0)),
        pl.BlockSpec((_B, 1), lambda: (0, 0)),
        pl.BlockSpec(memory_space=pltpu.SMEM),
    ],
    out_specs=pl.BlockSpec((_B, 1), lambda: (0, 0)),
    out_shape=jax.ShapeDtypeStruct((_B, 1), jnp.float32),
)


def kernel(x, tokens, top_token_ids, W, b, general_temp):
    bounds = jnp.searchsorted(
        top_token_ids,
        jnp.arange(_NT + 1, dtype=jnp.int32) * _TILE).astype(jnp.int32)
    inv_g = 1.0 / general_temp
    log2e = 1.4426950408889634
    bg = jnp.stack([b[0], inv_g, inv_g * log2e]).astype(jnp.float32)
    tok2d = tokens.reshape(_B, 1)

    m_all, s_all, topv, xtok = _tc_stream(
        x, top_token_ids, bounds, tok2d, bg)
    out = _tc_epi(topv, W, top_token_ids.reshape(1, _K), tok2d,
                  xtok, m_all, s_all, bg)
    return out.reshape(_B)


# speedup vs baseline: 1.0724x; 1.0724x over previous
import functools

import jax
import jax.numpy as jnp
from jax import lax
from jax.experimental import pallas as pl
from jax.experimental.pallas import tpu as pltpu
from jax.experimental.pallas import tpu_sc as plsc

_B = 128
_V = 100000
_K = 1024


_NC = 2
_NS = 16
_NW = _NC * _NS

_NTOT = _B * _K + _B
_PER_W = -(-_NTOT // _NW)
_PER_W += (-_PER_W) % 8
_NPAD = _PER_W * _NW


def _sc_gather_body(x_hbm, idx_hbm, out_hbm, idx_v, val_v, sem):
    wid = lax.axis_index("s") * _NC + lax.axis_index("c")
    base = wid * _PER_W
    pltpu.sync_copy(idx_hbm.at[pl.ds(base, _PER_W)], idx_v)
    pltpu.async_copy(x_hbm.at[idx_v], val_v, sem).wait()
    pltpu.sync_copy(val_v, out_hbm.at[pl.ds(base, _PER_W)])


@functools.cache
def _sc_gather():
    return pl.kernel(
        _sc_gather_body,
        out_type=jax.ShapeDtypeStruct((_NPAD,), jnp.float32),
        mesh=plsc.VectorSubcoreMesh(
            core_axis_name="c", subcore_axis_name="s",
            num_cores=_NC, num_subcores=_NS),
        scratch_types=[
            pltpu.VMEM((_PER_W,), jnp.int32),
            pltpu.VMEM((_PER_W,), jnp.float32),
            pltpu.SemaphoreType.DMA,
        ],
    )


_TILE = 16384
_NT = -(-_V // _TILE)


def _tc_stream_body(x_ref, bg_ref, m_out, s_out, m_s, s_s):
    i = pl.program_id(0)
    c = bg_ref[2]

    @pl.when(i == 0)
    def _init():
        m_s[...] = jnp.full((_B, 128), -jnp.inf, jnp.float32)
        s_s[...] = jnp.zeros((_B, 128), jnp.float32)

    @pl.when(i == _NT - 1)
    def _mask_tail():
        x_ref[:, _V % _TILE:] = jnp.full(
            (_B, _TILE - _V % _TILE), -3.0e38, jnp.float32)

    a = x_ref[...] * c
    m_old = m_s[...][:, :1]
    s_old = s_s[...][:, :1]
    m_new = jnp.maximum(m_old, jnp.max(a, axis=1, keepdims=True))
    s_new = s_old * jnp.exp2(m_old - m_new) + jnp.sum(
        jnp.exp2(a - m_new), axis=1, keepdims=True)
    m_s[...] = jnp.broadcast_to(m_new, (_B, 128))
    s_s[...] = jnp.broadcast_to(s_new, (_B, 128))

    @pl.when(i == _NT - 1)
    def _emit():
        m_out[...] = m_new
        s_out[...] = s_new


_tc_stream = pl.pallas_call(
    _tc_stream_body,
    grid=(_NT,),
    in_specs=[
        pl.BlockSpec((_B, _TILE), lambda i: (0, i)),
        pl.BlockSpec(memory_space=pltpu.SMEM),
    ],
    out_specs=[
        pl.BlockSpec((_B, 1), lambda i: (0, 0)),
        pl.BlockSpec((_B, 1), lambda i: (0, 0)),
    ],
    out_shape=[
        jax.ShapeDtypeStruct((_B, 1), jnp.float32),
        jax.ShapeDtypeStruct((_B, 1), jnp.float32),
    ],
    scratch_shapes=[
        pltpu.VMEM((_B, 128), jnp.float32),
        pltpu.VMEM((_B, 128), jnp.float32),
    ],
    compiler_params=pltpu.CompilerParams(
        dimension_semantics=("arbitrary",)),
)


def _tc_epi_body(top_ref, w_ref, xtok_ref, intop_ref, m_ref, s_ref, bg_ref,
                 out_ref):
    inv_g = bg_ref[1]
    c = bg_ref[2]
    a_top = top_ref[...] * inv_g
    a_bf = a_top.astype(jnp.bfloat16).astype(jnp.float32)
    w_bf = w_ref[...].astype(jnp.bfloat16).astype(jnp.float32)
    t = jnp.clip(
        jnp.sum(a_bf * w_bf, axis=1, keepdims=True) + bg_ref[0],
        1e-6, None)
    a2_top = top_ref[...] * c
    top_mod = a2_top / t
    m_all = m_ref[...]
    s_all = s_ref[...]
    m2 = jnp.maximum(m_all, jnp.max(top_mod, axis=1, keepdims=True))
    corr = jnp.sum(jnp.exp2(top_mod - m2) - jnp.exp2(a2_top - m2),
                   axis=1, keepdims=True)
    s_tot = s_all * jnp.exp2(m_all - m2) + corr
    a_tok = xtok_ref[...] * c
    a_eff = jnp.where(intop_ref[...] != 0, a_tok / t, a_tok)
    out_ref[...] = jnp.exp2(a_eff - m2) / s_tot


_tc_epi = pl.pallas_call(
    _tc_epi_body,
    in_specs=[
        pl.BlockSpec((_B, _K), lambda: (0, 0)),
        pl.BlockSpec((1, _K), lambda: (0, 0)),
        pl.BlockSpec((_B, 1), lambda: (0, 0)),
        pl.BlockSpec((_B, 1), lambda: (0, 0)),
        pl.BlockSpec((_B, 1), lambda: (0, 0)),
        pl.BlockSpec((_B, 1), lambda: (0, 0)),
        pl.BlockSpec(memory_space=pltpu.SMEM),
    ],
    out_specs=pl.BlockSpec((_B, 1), lambda: (0, 0)),
    out_shape=jax.ShapeDtypeStruct((_B, 1), jnp.float32),
)


def kernel(x, tokens, top_token_ids, W, b, general_temp):
    flat_top = (jnp.arange(_B, dtype=jnp.int32)[:, None] * _V
                + top_token_ids[None, :]).reshape(-1)
    flat_tok = jnp.arange(_B, dtype=jnp.int32) * _V + tokens
    idx = jnp.concatenate([
        flat_top, flat_tok,
        jnp.zeros((_NPAD - _NTOT,), jnp.int32)])

    pos = jnp.searchsorted(top_token_ids, tokens)
    in_top = ((pos < _K)
              & (top_token_ids[jnp.minimum(pos, _K - 1)] == tokens))

    inv_g = 1.0 / general_temp
    log2e = 1.4426950408889634
    bg = jnp.stack([b[0], inv_g, inv_g * log2e]).astype(jnp.float32)

    gathered = _sc_gather()(x.reshape(-1), idx)
    top = gathered[: _B * _K].reshape(_B, _K)
    xtok = gathered[_B * _K: _B * _K + _B].reshape(_B, 1)

    m_all, s_all = _tc_stream(x, bg)
    out = _tc_epi(top, W, xtok,
                  in_top.astype(jnp.int32).reshape(_B, 1),
                  m_all, s_all, bg)
    return out.reshape(_B)
